# baseline (speedup 1.0000x reference)
import jax
import jax.numpy as jnp
from jax.experimental import pallas as pl
from jax.experimental.pallas import tpu as pltpu

N = 1024
D = 64
K = 16
R = 256
GRID = N // R


def _dist_and_keys(norm_ref, af_ref, bf_ref, i):
    a_blk = af_ref[pl.ds(i * R, R), :]
    b_all = bf_ref[...]
    sq = jax.lax.dot_general(a_blk, b_all, (((1,), (1,)), ((), ())),
                             preferred_element_type=jnp.float32,
                             precision=jax.lax.Precision.HIGHEST)
    dist = jnp.sqrt(jnp.maximum(sq, 0.0))
    dist = dist * (1.0 / norm_ref[0, 0])

    rows = i * R + jax.lax.broadcasted_iota(jnp.int32, (R, N), 0)
    cols = jax.lax.broadcasted_iota(jnp.int32, (R, N), 1)
    diag = rows == cols
    dist = jnp.where(diag, 0.0, dist)

    kf = jnp.where(diag, jnp.float32(jnp.inf), dist)
    return dist, kf


def _body(norm_ref, lat_ref, dx_ref, mx_ref, mxc_ref,
          dist_ref, nm_ref, o12_ref, o21_ref,
          af_ref, bf_ref, dist_s_ref, tcol_ref, trow_ref, acc_ref):
    p = pl.program_id(0)
    i = pl.program_id(1)
    inf = jnp.float32(jnp.inf)

    @pl.when(jnp.logical_and(p == 0, i == 0))
    def _build_augmented():
        x = lat_ref[...]
        r = jnp.sum(x * x, axis=1)[:, None]
        one = jnp.ones((N, 1), jnp.float32)
        af_ref[:, pl.ds(0, D)] = x
        af_ref[:, pl.ds(D, 1)] = r
        af_ref[:, pl.ds(D + 1, 1)] = one
        bf_ref[:, pl.ds(0, D)] = -2.0 * x
        bf_ref[:, pl.ds(D, 1)] = one
        bf_ref[:, pl.ds(D + 1, 1)] = r

    @pl.when(p == 0)
    def _phase_a():
        dist, kf = _dist_and_keys(norm_ref, af_ref, bf_ref, i)
        dist_s_ref[pl.ds(i * R, R), :] = dist
        for t in range(K):
            m = jnp.min(kf, axis=1, keepdims=True)
            if t < K - 1:
                kf = jnp.where(kf == m, inf, kf)
        tcol_ref[pl.ds(i * R, R), :] = m
        ii = jax.lax.broadcasted_iota(jnp.int32, (R, R), 0)
        jj = jax.lax.broadcasted_iota(jnp.int32, (R, R), 1)
        eye = jnp.where(ii == jj, 1.0, 0.0)
        mrow = jax.lax.dot_general(m, eye, (((0,), (0,)), ((), ())),
                                   preferred_element_type=jnp.float32)
        trow_ref[:, pl.ds(i * R, R)] = mrow

        @pl.when(i == 0)
        def _():
            acc_ref[0] = 0.0
            acc_ref[1] = 0.0
            acc_ref[2] = 0.0
            acc_ref[3] = 0.0

    @pl.when(p == 1)
    def _phase_b():
        dist = dist_s_ref[pl.ds(i * R, R), :]
        rows = i * R + jax.lax.broadcasted_iota(jnp.int32, (R, N), 0)
        cols = jax.lax.broadcasted_iota(jnp.int32, (R, N), 1)
        kf = jnp.where(rows == cols, inf, dist)
        tcol = tcol_ref[pl.ds(i * R, R), :]
        trow = trow_ref[...]

        zr = kf <= tcol
        zc = kf <= trow
        ztot = jnp.where(jnp.logical_or(zr, zc), 1.0, 0.0)

        dx = dx_ref[...]
        mx = mx_ref[...]
        v1 = mx * dx - mx * dist
        s1 = jnp.sum(v1 * v1)
        dd = dx - dist
        v2 = jnp.where(zr, dd, 0.0)
        s2 = jnp.sum(v2 * v2)

        c = jnp.where(mx != 0.0, 1.0, 0.0)
        zc_op = ztot * c
        cc = jnp.where(mxc_ref[...] != 0.0, 1.0, 0.0)
        prod = jax.lax.dot_general(
            zc_op.astype(jnp.bfloat16), cc.astype(jnp.bfloat16),
            (((1,), (0,)), ((), ())), preferred_element_type=jnp.float32)
        ii = jax.lax.broadcasted_iota(jnp.int32, (R, R), 0)
        jj = jax.lax.broadcasted_iota(jnp.int32, (R, R), 1)
        t_zcc = jnp.sum(jnp.where(ii == jj, prod, 0.0))

        acc_ref[0] += s1
        acc_ref[1] += s2
        acc_ref[2] += jnp.sum(ztot)
        acc_ref[3] += 2.0 * jnp.sum(zc_op) - t_zcc

        @pl.when(i == GRID - 1)
        def _():
            s1t = acc_ref[0]
            s2t = acc_ref[1]
            o12_ref[0, 0] = s1t
            o21_ref[0, 0] = s2t
            dist_ref[0, 0] = s1t + s2t
            nm_ref[0, 0] = (acc_ref[2] - acc_ref[3]) / acc_ref[2]


@jax.jit
def kernel(latent, latent_norm, dist_X, pair_mask_X):
    norm2d = latent_norm.reshape(1, 1)

    distance, nm, o12, o21 = pl.pallas_call(
        _body,
        grid=(2, GRID),
        in_specs=[
            pl.BlockSpec(memory_space=pltpu.SMEM),
            pl.BlockSpec((N, D), lambda p, i: (0, 0)),
            pl.BlockSpec((R, N), lambda p, i: (i * p, 0)),
            pl.BlockSpec((R, N), lambda p, i: (i * p, 0)),
            pl.BlockSpec((N, R), lambda p, i: (0, i * p)),
        ],
        out_specs=[
            pl.BlockSpec(memory_space=pltpu.SMEM),
            pl.BlockSpec(memory_space=pltpu.SMEM),
            pl.BlockSpec(memory_space=pltpu.SMEM),
            pl.BlockSpec(memory_space=pltpu.SMEM),
        ],
        out_shape=[
            jax.ShapeDtypeStruct((1, 1), jnp.float32),
            jax.ShapeDtypeStruct((1, 1), jnp.float32),
            jax.ShapeDtypeStruct((1, 1), jnp.float32),
            jax.ShapeDtypeStruct((1, 1), jnp.float32),
        ],
        scratch_shapes=[
            pltpu.VMEM((N, D + 2), jnp.float32),
            pltpu.VMEM((N, D + 2), jnp.float32),
            pltpu.VMEM((N, N), jnp.float32),
            pltpu.VMEM((N, 1), jnp.float32),
            pltpu.VMEM((1, N), jnp.float32),
            pltpu.SMEM((4,), jnp.float32),
        ],
    )(norm2d, latent, dist_X, pair_mask_X, pair_mask_X)

    return (distance.reshape(()), nm.reshape(()),
            o12.reshape(()), o21.reshape(()))
